# baseline (speedup 1.0000x reference)
import jax
import jax.numpy as jnp
from jax.experimental import pallas as pl

B, N = 128, 64
ATOM_DIM, HYBRID_DIM, CONT_DIM, CONT_IN = 32, 16, 16, 16
NODE_DIM, EDGE_DIM, MESS_DIM, TIME_DIM = 64, 32, 64, 64
ATOM_VOCAB, HYBRID_VOCAB, BOND_VOCAB = 100, 8, 8

IB = 1

_LOG1E4 = 9.210340371976184


def _prologue_kernel(
    times_ref, cont_ref, at_ref, hy_ref,
    atom_tab_ref, hyb_tab_ref, bond_tab_ref,
    W_t1_ref, b_t1_ref, W_tn_ref, b_tn_ref, W_te_ref, b_te_ref,
    W_cont_ref, b_cont_ref, W_node_ref, b_node_ref,
    W_edge_ref, b_edge_ref, W_msg_ref, b_msg_ref, W_eupd_ref, b_eupd_ref,
    nodes_out, a_out, c_out, a2_out, c2_out, tab_out, te_out,
):
    f32 = jnp.float32
    half = TIME_DIM // 2
    k = jax.lax.broadcasted_iota(jnp.int32, (1, half), 1).astype(f32)
    freqs = jnp.exp(-_LOG1E4 * k / half)
    args = times_ref[...] * freqs
    t = jnp.concatenate([jnp.sin(args), jnp.cos(args)], axis=1)
    h = jax.nn.gelu(
        jnp.dot(t, W_t1_ref[...], preferred_element_type=f32) + b_t1_ref[...])
    t_nodes = jnp.dot(h, W_tn_ref[...], preferred_element_type=f32) + b_tn_ref[...]
    t_edges = jnp.dot(h, W_te_ref[...], preferred_element_type=f32) + b_te_ref[...]
    te_out[...] = t_edges

    cont_h = jax.nn.gelu(
        jnp.dot(cont_ref[...], W_cont_ref[...], preferred_element_type=f32)
        + b_cont_ref[...])
    at = at_ref[...]
    oh_a = (at == jax.lax.broadcasted_iota(jnp.int32, (B * N, ATOM_VOCAB), 1)
            ).astype(f32)
    aemb = jnp.dot(oh_a, atom_tab_ref[...], preferred_element_type=f32)
    hy = hy_ref[...]
    oh_h = (hy == jax.lax.broadcasted_iota(jnp.int32, (B * N, HYBRID_VOCAB), 1)
            ).astype(f32)
    hemb = jnp.dot(oh_h, hyb_tab_ref[...], preferred_element_type=f32)
    nfeat = jnp.concatenate([aemb, hemb, cont_h], axis=1)
    nodes0 = jax.nn.gelu(
        jnp.dot(nfeat, W_node_ref[...], preferred_element_type=f32)
        + b_node_ref[...])
    nodes = (nodes0.reshape(B, N, NODE_DIM) + t_nodes[:, None, :]
             ).reshape(B * N, NODE_DIM)
    nodes_out[...] = nodes

    e_table = jax.nn.gelu(
        jnp.dot(bond_tab_ref[...], W_edge_ref[...], preferred_element_type=f32)
        + b_edge_ref[...])
    W_msg = W_msg_ref[...]
    Wm_i, Wm_j, Wm_e = W_msg[:64], W_msg[64:128], W_msg[128:160]
    W_eu = W_eupd_ref[...]
    Wu_i, Wu_j, Wu_e = W_eu[:64], W_eu[64:128], W_eu[128:160]
    em_table = jnp.dot(e_table, Wm_e, preferred_element_type=f32)
    eu_table = jnp.dot(e_table, Wu_e, preferred_element_type=f32)
    tab_out[...] = jnp.concatenate([em_table, eu_table, e_table], axis=1)

    te_m = jnp.dot(t_edges, Wm_e, preferred_element_type=f32) + b_msg_ref[...]
    te_u = jnp.dot(t_edges, Wu_e, preferred_element_type=f32) + b_eupd_ref[...]
    a = (jnp.dot(nodes, Wm_i, preferred_element_type=f32).reshape(B, N, MESS_DIM)
         + te_m[:, None, :]).reshape(B * N, MESS_DIM)
    a2 = (jnp.dot(nodes, Wu_i, preferred_element_type=f32).reshape(B, N, EDGE_DIM)
          + te_u[:, None, :]).reshape(B * N, EDGE_DIM)
    a_out[...] = a
    c_out[...] = jnp.dot(nodes, Wm_j, preferred_element_type=f32)
    a2_out[...] = a2
    c2_out[...] = jnp.dot(nodes, Wu_j, preferred_element_type=f32)


def _pairs_kernel(
    eT_ref, pmT_ref, act_ref, cct_ref, tabT_ref, teT_ref,
    aggT_out, neT_out,
):
    f32 = jnp.float32
    et = eT_ref[...]
    tabT = tabT_ref[...]
    X = act_ref[...][:, None, :, :] + cct_ref[...][None, :, :, :]
    for k in range(BOND_VOCAB):
        mk = (et == k).astype(f32)[:, :, None, :]
        X = X + mk * tabT[None, None, :, k:k + 1]
    G = jax.nn.gelu(X[:, :, :96, :])
    pm4 = pmT_ref[...][:, :, None, :]
    msgT = G[:, :, :MESS_DIM, :] * pm4
    aggT_out[...] = jnp.sum(msgT, axis=1)
    neT_out[...] = (G[:, :, MESS_DIM:, :]
                    + teT_ref[...][None, None, :, :]
                    + X[:, :, 96:, :]) * pm4


def _node_upd_kernel(nodes_ref, agg_ref, nm_ref, W_ref, b_ref, out_ref):
    f32 = jnp.float32
    x = jnp.concatenate([nodes_ref[...], agg_ref[...]], axis=1)
    nn = jax.nn.gelu(
        jnp.dot(x, W_ref[...], preferred_element_type=f32) + b_ref[...])
    out_ref[...] = nn * nm_ref[...]


def kernel(atom_type, hybrid, cont, edges, node_mask, pair_mask, times,
           atom_table, hybrid_table, bond_table,
           W_t1, b_t1, W_tn, b_tn, W_te, b_te, W_cont, b_cont,
           W_node, b_node, W_edge, b_edge, W_msg, b_msg,
           W_upd, b_upd, W_eupd, b_eupd):
    f32 = jnp.float32
    r2 = lambda v: v.reshape(1, -1)

    nodes, a, c, a2, c2, tab, t_edges = pl.pallas_call(
        _prologue_kernel,
        out_shape=(
            jax.ShapeDtypeStruct((B * N, NODE_DIM), f32),
            jax.ShapeDtypeStruct((B * N, MESS_DIM), f32),
            jax.ShapeDtypeStruct((B * N, MESS_DIM), f32),
            jax.ShapeDtypeStruct((B * N, EDGE_DIM), f32),
            jax.ShapeDtypeStruct((B * N, EDGE_DIM), f32),
            jax.ShapeDtypeStruct((BOND_VOCAB, 128), f32),
            jax.ShapeDtypeStruct((B, EDGE_DIM), f32),
        ),
    )(
        times.reshape(B, 1), cont.reshape(B * N, CONT_IN),
        atom_type.reshape(B * N, 1), hybrid.reshape(B * N, 1),
        atom_table, hybrid_table, bond_table,
        W_t1, r2(b_t1), W_tn, r2(b_tn), W_te, r2(b_te),
        W_cont, r2(b_cont), W_node, r2(b_node),
        W_edge, r2(b_edge), W_msg, r2(b_msg), W_eupd, r2(b_eupd),
    )

    z32 = jnp.zeros((B * N, 32), f32)
    ACT = (jnp.concatenate([a, a2, z32], axis=1)
           .reshape(B, N, 128).transpose(1, 2, 0))
    CCT = (jnp.concatenate([c, c2, z32], axis=1)
           .reshape(B, N, 128).transpose(1, 2, 0))
    eT = edges.transpose(1, 2, 0)
    pmT = pair_mask.transpose(1, 2, 0)
    tabT = tab.T
    teT = t_edges.T

    grid = (N // IB,)
    bspec = lambda *blk: pl.BlockSpec(blk, lambda i: (i,) + (0,) * (len(blk) - 1))
    full = lambda *shp: pl.BlockSpec(shp, lambda i: (0,) * len(shp))

    aggT, neT = pl.pallas_call(
        _pairs_kernel,
        grid=grid,
        in_specs=[
            bspec(IB, N, B),
            bspec(IB, N, B),
            bspec(IB, 128, B),
            full(N, 128, B),
            full(128, BOND_VOCAB),
            full(EDGE_DIM, B),
        ],
        out_specs=(
            bspec(IB, MESS_DIM, B),
            bspec(IB, N, EDGE_DIM, B),
        ),
        out_shape=(
            jax.ShapeDtypeStruct((N, MESS_DIM, B), f32),
            jax.ShapeDtypeStruct((N, N, EDGE_DIM, B), f32),
        ),
    )(eT, pmT, ACT, CCT, tabT, teT)

    agg = aggT.transpose(2, 0, 1).reshape(B * N, MESS_DIM)
    nn = pl.pallas_call(
        _node_upd_kernel,
        out_shape=jax.ShapeDtypeStruct((B * N, NODE_DIM), f32),
    )(nodes, agg, node_mask.reshape(B * N, 1), W_upd, r2(b_upd))

    new_nodes = nn.reshape(B, N, NODE_DIM)
    new_edges = neT.transpose(3, 0, 1, 2)
    return new_nodes, new_edges


# speedup vs baseline: 2.5193x; 2.5193x over previous
import jax
import jax.numpy as jnp
from jax.experimental import pallas as pl

B, N = 128, 64
ATOM_DIM, HYBRID_DIM, CONT_DIM, CONT_IN = 32, 16, 16, 16
NODE_DIM, EDGE_DIM, MESS_DIM, TIME_DIM = 64, 32, 64, 64
ATOM_VOCAB, HYBRID_VOCAB, BOND_VOCAB = 100, 8, 8

IB = 1

_LOG1E4 = 9.210340371976184


def _prologue_kernel(
    times_ref, cont_ref, at_ref, hy_ref,
    atom_tab_ref, hyb_tab_ref, bond_tab_ref,
    W_t1_ref, b_t1_ref, W_tn_ref, b_tn_ref, W_te_ref, b_te_ref,
    W_cont_ref, b_cont_ref, W_node_ref, b_node_ref,
    W_edge_ref, b_edge_ref, W_msg_ref, b_msg_ref, W_eupd_ref, b_eupd_ref,
    nodes_out, a_out, c_out, a2_out, c2_out, tab_out, te_out,
):
    f32 = jnp.float32
    half = TIME_DIM // 2
    k = jax.lax.broadcasted_iota(jnp.int32, (1, half), 1).astype(f32)
    freqs = jnp.exp(-_LOG1E4 * k / half)
    args = times_ref[...] * freqs
    t = jnp.concatenate([jnp.sin(args), jnp.cos(args)], axis=1)
    h = jax.nn.gelu(
        jnp.dot(t, W_t1_ref[...], preferred_element_type=f32) + b_t1_ref[...])
    t_nodes = jnp.dot(h, W_tn_ref[...], preferred_element_type=f32) + b_tn_ref[...]
    t_edges = jnp.dot(h, W_te_ref[...], preferred_element_type=f32) + b_te_ref[...]
    te_out[...] = t_edges

    cont_h = jax.nn.gelu(
        jnp.dot(cont_ref[...], W_cont_ref[...], preferred_element_type=f32)
        + b_cont_ref[...])
    at = at_ref[...]
    oh_a = (at == jax.lax.broadcasted_iota(jnp.int32, (B * N, ATOM_VOCAB), 1)
            ).astype(f32)
    aemb = jnp.dot(oh_a, atom_tab_ref[...], preferred_element_type=f32)
    hy = hy_ref[...]
    oh_h = (hy == jax.lax.broadcasted_iota(jnp.int32, (B * N, HYBRID_VOCAB), 1)
            ).astype(f32)
    hemb = jnp.dot(oh_h, hyb_tab_ref[...], preferred_element_type=f32)
    nfeat = jnp.concatenate([aemb, hemb, cont_h], axis=1)
    nodes0 = jax.nn.gelu(
        jnp.dot(nfeat, W_node_ref[...], preferred_element_type=f32)
        + b_node_ref[...])
    nodes = (nodes0.reshape(B, N, NODE_DIM) + t_nodes[:, None, :]
             ).reshape(B * N, NODE_DIM)
    nodes_out[...] = nodes

    e_table = jax.nn.gelu(
        jnp.dot(bond_tab_ref[...], W_edge_ref[...], preferred_element_type=f32)
        + b_edge_ref[...])
    W_msg = W_msg_ref[...]
    Wm_i, Wm_j, Wm_e = W_msg[:64], W_msg[64:128], W_msg[128:160]
    W_eu = W_eupd_ref[...]
    Wu_i, Wu_j, Wu_e = W_eu[:64], W_eu[64:128], W_eu[128:160]
    em_table = jnp.dot(e_table, Wm_e, preferred_element_type=f32)
    eu_table = jnp.dot(e_table, Wu_e, preferred_element_type=f32)
    tab_out[...] = jnp.concatenate([em_table, eu_table, e_table], axis=1)

    te_m = jnp.dot(t_edges, Wm_e, preferred_element_type=f32) + b_msg_ref[...]
    te_u = jnp.dot(t_edges, Wu_e, preferred_element_type=f32) + b_eupd_ref[...]
    a = (jnp.dot(nodes, Wm_i, preferred_element_type=f32).reshape(B, N, MESS_DIM)
         + te_m[:, None, :]).reshape(B * N, MESS_DIM)
    a2 = (jnp.dot(nodes, Wu_i, preferred_element_type=f32).reshape(B, N, EDGE_DIM)
          + te_u[:, None, :]).reshape(B * N, EDGE_DIM)
    a_out[...] = a
    c_out[...] = jnp.dot(nodes, Wm_j, preferred_element_type=f32)
    a2_out[...] = a2
    c2_out[...] = jnp.dot(nodes, Wu_j, preferred_element_type=f32)


def _pairs_kernel(
    eT_ref, pmT_ref, act_ref, cct_ref, tabT_ref, teT_ref,
    aggT_out, neT_out,
):
    f32 = jnp.float32
    e2 = eT_ref[...][0]
    tabT = tabT_ref[...]
    oh3 = (e2[None, :, :] == jax.lax.broadcasted_iota(
        jnp.int32, (BOND_VOCAB, N, B), 0)).astype(f32)
    gathT = jnp.stack(
        [jnp.dot(tabT, oh3[:, j, :], preferred_element_type=f32)
         for j in range(N)], axis=0)[None]
    X = act_ref[...][:, None, :, :] + cct_ref[...][None, :, :, :] + gathT
    G = jax.nn.gelu(X[:, :, :96, :])
    pm4 = pmT_ref[...][:, :, None, :]
    msgT = G[:, :, :MESS_DIM, :] * pm4
    aggT_out[...] = jnp.sum(msgT, axis=1)
    neT_out[...] = (G[:, :, MESS_DIM:, :]
                    + teT_ref[...][None, None, :, :]
                    + X[:, :, 96:, :]) * pm4


def _node_upd_kernel(nodes_ref, agg_ref, nm_ref, W_ref, b_ref, out_ref):
    f32 = jnp.float32
    x = jnp.concatenate([nodes_ref[...], agg_ref[...]], axis=1)
    nn = jax.nn.gelu(
        jnp.dot(x, W_ref[...], preferred_element_type=f32) + b_ref[...])
    out_ref[...] = nn * nm_ref[...]


def kernel(atom_type, hybrid, cont, edges, node_mask, pair_mask, times,
           atom_table, hybrid_table, bond_table,
           W_t1, b_t1, W_tn, b_tn, W_te, b_te, W_cont, b_cont,
           W_node, b_node, W_edge, b_edge, W_msg, b_msg,
           W_upd, b_upd, W_eupd, b_eupd):
    f32 = jnp.float32
    r2 = lambda v: v.reshape(1, -1)

    nodes, a, c, a2, c2, tab, t_edges = pl.pallas_call(
        _prologue_kernel,
        out_shape=(
            jax.ShapeDtypeStruct((B * N, NODE_DIM), f32),
            jax.ShapeDtypeStruct((B * N, MESS_DIM), f32),
            jax.ShapeDtypeStruct((B * N, MESS_DIM), f32),
            jax.ShapeDtypeStruct((B * N, EDGE_DIM), f32),
            jax.ShapeDtypeStruct((B * N, EDGE_DIM), f32),
            jax.ShapeDtypeStruct((BOND_VOCAB, 128), f32),
            jax.ShapeDtypeStruct((B, EDGE_DIM), f32),
        ),
    )(
        times.reshape(B, 1), cont.reshape(B * N, CONT_IN),
        atom_type.reshape(B * N, 1), hybrid.reshape(B * N, 1),
        atom_table, hybrid_table, bond_table,
        W_t1, r2(b_t1), W_tn, r2(b_tn), W_te, r2(b_te),
        W_cont, r2(b_cont), W_node, r2(b_node),
        W_edge, r2(b_edge), W_msg, r2(b_msg), W_eupd, r2(b_eupd),
    )

    z32 = jnp.zeros((B * N, 32), f32)
    ACT = (jnp.concatenate([a, a2, z32], axis=1)
           .reshape(B, N, 128).transpose(1, 2, 0))
    CCT = (jnp.concatenate([c, c2, z32], axis=1)
           .reshape(B, N, 128).transpose(1, 2, 0))
    eT = edges.transpose(1, 2, 0)
    pmT = pair_mask.transpose(1, 2, 0)
    tabT = tab.T
    teT = t_edges.T

    grid = (N // IB,)
    bspec = lambda *blk: pl.BlockSpec(blk, lambda i: (i,) + (0,) * (len(blk) - 1))
    full = lambda *shp: pl.BlockSpec(shp, lambda i: (0,) * len(shp))

    aggT, neT = pl.pallas_call(
        _pairs_kernel,
        grid=grid,
        in_specs=[
            bspec(IB, N, B),
            bspec(IB, N, B),
            bspec(IB, 128, B),
            full(N, 128, B),
            full(128, BOND_VOCAB),
            full(EDGE_DIM, B),
        ],
        out_specs=(
            bspec(IB, MESS_DIM, B),
            bspec(IB, N, EDGE_DIM, B),
        ),
        out_shape=(
            jax.ShapeDtypeStruct((N, MESS_DIM, B), f32),
            jax.ShapeDtypeStruct((N, N, EDGE_DIM, B), f32),
        ),
    )(eT, pmT, ACT, CCT, tabT, teT)

    agg = aggT.transpose(2, 0, 1).reshape(B * N, MESS_DIM)
    nn = pl.pallas_call(
        _node_upd_kernel,
        out_shape=jax.ShapeDtypeStruct((B * N, NODE_DIM), f32),
    )(nodes, agg, node_mask.reshape(B * N, 1), W_upd, r2(b_upd))

    new_nodes = nn.reshape(B, N, NODE_DIM)
    new_edges = neT.transpose(3, 0, 1, 2)
    return new_nodes, new_edges
